# baseline (speedup 1.0000x reference)
import functools

import jax
import jax.numpy as jnp
from jax import lax
from jax.experimental import pallas as pl
from jax.experimental.pallas import tpu as pltpu
from jax.experimental.pallas import tpu_sc as plsc

BATCH = 16384
DIM_HEAD = 64
DIM_MID = 32
DIM_TAIL = 16
NUM_HEAD = 100000

ROW_MID = NUM_HEAD
ROW_TAIL = ROW_MID + NUM_HEAD // 2

_INFO = plsc.get_sparse_core_info()
NC, NS = _INFO.num_cores, _INFO.num_subcores
NW = NC * NS
N_PER = BATCH // NW
CH = 32
NSTEP = N_PER // 16


def _body(x_hbm, g_hbm, tbl_hbm, out_hbm,
          xv, gv, xk0, pk0, xk1, pk1, sk1, xk2, pk2, sk2,
          gb0, gb1, obuf, sem0, sem1, sem2):
    wid = lax.axis_index("s") * NC + lax.axis_index("c")
    base = wid * N_PER
    pltpu.sync_copy(x_hbm.at[pl.ds(base, N_PER)], xv)
    pltpu.sync_copy(g_hbm.at[pl.ds(base, N_PER)], gv)

    zi = jnp.zeros((16,), jnp.int32)
    zf = jnp.zeros((16,), jnp.float32)

    for i in range(NSTEP):
        xk0[pl.ds(i * 16, 16)] = zi
        xk1[pl.ds(i * 16, 16)] = zi
        xk2[pl.ds(i * 16, 16)] = zi

    iota = lax.iota(jnp.int32, 16)
    offs = [jnp.int32(0), jnp.int32(0), jnp.int32(0)]
    for c in range(NSTEP):
        xc = xv[pl.ds(c * 16, 16)]
        gc = gv[pl.ds(c * 16, 16)]
        posc = iota + (c * 16)
        rows_g = (xc,
                  ROW_MID + lax.shift_right_logical(xc, 1),
                  ROW_TAIL + lax.shift_right_logical(xc, 2))
        subs_g = (None,
                  lax.shift_left(lax.bitwise_and(xc, 1), 5),
                  lax.shift_left(lax.bitwise_and(xc, 3), 4))
        for k, (xk, pk, sk) in enumerate(((xk0, pk0, None),
                                          (xk1, pk1, sk1),
                                          (xk2, pk2, sk2))):
            m = gc == k
            ones = m.astype(jnp.int32)
            incl = plsc.cumsum(ones)
            dest = offs[k] + incl - ones
            plsc.store_scatter(xk, [dest], rows_g[k], mask=m)
            plsc.store_scatter(pk, [dest], posc, mask=m)
            if sk is not None:
                plsc.store_scatter(sk, [dest], subs_g[k], mask=m)
            offs[k] = offs[k] + jnp.sum(ones)

    def fire_all(nk, xk, gbuf, sem):
        trips = lax.shift_right_logical(nk + (CH - 1), 5)

        def fire(j, carry):
            pltpu.async_copy(tbl_hbm.at[xk.at[pl.ds(j * CH, CH)]],
                             gbuf.at[pl.ds(j * CH, CH)], sem)
            return carry

        lax.fori_loop(0, trips, fire, jnp.int32(0))
        return trips

    def drain_all(trips, xk, gbuf, sem):
        def drain(j, carry):
            pltpu.make_async_copy(tbl_hbm.at[xk.at[pl.ds(j * CH, CH)]],
                                  gbuf.at[pl.ds(j * CH, CH)], sem).wait()
            return carry

        lax.fori_loop(0, trips, drain, jnp.int32(0))

    t0 = fire_all(offs[0], xk0, gb0, sem0)
    t1 = fire_all(offs[1], xk1, gb1, sem1)

    def reloc(nk, pk, emit16):
        nsteps = lax.shift_right_logical(nk + 15, 4)

        def step(j, carry):
            rows = iota + j * 16
            mask = rows < nk
            pos = plsc.load_gather(pk, [rows])
            pos = lax.bitwise_and(pos, N_PER - 1)
            emit16(rows, pos, mask)
            return carry

        lax.fori_loop(0, nsteps, step, jnp.int32(0))

    def head16(rows, pos, mask):
        for c in range(DIM_HEAD):
            cv = jnp.full((16,), c, jnp.int32)
            v = plsc.load_gather(gb0, [rows, cv])
            plsc.store_scatter(obuf, [pos, cv], v, mask=mask)

    def mid16(rows, pos, mask):
        sub = plsc.load_gather(sk1, [rows])
        sub = lax.bitwise_and(sub, DIM_MID)
        for c in range(DIM_MID):
            cv = jnp.full((16,), c, jnp.int32)
            v = plsc.load_gather(gb1, [rows, cv + sub])
            plsc.store_scatter(obuf, [pos, cv], v, mask=mask)
        for c in range(DIM_MID, DIM_HEAD):
            cv = jnp.full((16,), c, jnp.int32)
            plsc.store_scatter(obuf, [pos, cv], zf, mask=mask)

    def tail16(rows, pos, mask):
        sub = plsc.load_gather(sk2, [rows])
        sub = lax.bitwise_and(sub, DIM_HEAD - DIM_TAIL)
        for c in range(DIM_TAIL):
            cv = jnp.full((16,), c, jnp.int32)
            v = plsc.load_gather(gb0, [rows, cv + sub])
            for q in range(DIM_HEAD // DIM_TAIL):
                cq = jnp.full((16,), c + q * DIM_TAIL, jnp.int32)
                plsc.store_scatter(obuf, [pos, cq], v, mask=mask)

    drain_all(t0, xk0, gb0, sem0)
    reloc(offs[0], pk0, head16)
    t2 = fire_all(offs[2], xk2, gb0, sem2)
    drain_all(t1, xk1, gb1, sem1)
    reloc(offs[1], pk1, mid16)
    drain_all(t2, xk2, gb0, sem2)
    reloc(offs[2], pk2, tail16)

    pltpu.sync_copy(obuf, out_hbm.at[pl.ds(base, N_PER)])


@jax.jit
def _sc_lookup(x, g, tbl):
    mesh = plsc.VectorSubcoreMesh(core_axis_name="c", subcore_axis_name="s")
    f = functools.partial(
        pl.kernel,
        mesh=mesh,
        compiler_params=pltpu.CompilerParams(
            needs_layout_passes=False, use_tc_tiling_on_sc=False),
        out_type=jax.ShapeDtypeStruct((BATCH, DIM_HEAD), jnp.float32),
        scratch_types=[
            pltpu.VMEM((N_PER,), jnp.int32),
            pltpu.VMEM((N_PER,), jnp.int32),
            pltpu.VMEM((N_PER,), jnp.int32),
            pltpu.VMEM((N_PER,), jnp.int32),
            pltpu.VMEM((N_PER,), jnp.int32),
            pltpu.VMEM((N_PER,), jnp.int32),
            pltpu.VMEM((N_PER,), jnp.int32),
            pltpu.VMEM((N_PER,), jnp.int32),
            pltpu.VMEM((N_PER,), jnp.int32),
            pltpu.VMEM((N_PER,), jnp.int32),
            pltpu.VMEM((N_PER, DIM_HEAD), jnp.float32),
            pltpu.VMEM((N_PER, DIM_HEAD), jnp.float32),
            pltpu.VMEM((N_PER, DIM_HEAD), jnp.float32),
            pltpu.SemaphoreType.DMA,
            pltpu.SemaphoreType.DMA,
            pltpu.SemaphoreType.DMA,
        ],
    )(_body)
    return f(x, g, tbl)


def kernel(x, frequency_groups, head_table, mid_table, tail_table):
    tbl = jnp.concatenate(
        [head_table,
         mid_table[:NUM_HEAD].reshape(NUM_HEAD // 2, DIM_HEAD),
         tail_table.reshape(NUM_HEAD // 4, DIM_HEAD)], axis=0)
    return _sc_lookup(x.astype(jnp.int32), frequency_groups.astype(jnp.int32),
                      tbl)


# speedup vs baseline: 1.4687x; 1.4687x over previous
import functools

import jax
import jax.numpy as jnp
from jax import lax
from jax.experimental import pallas as pl
from jax.experimental.pallas import tpu as pltpu
from jax.experimental.pallas import tpu_sc as plsc

BATCH = 16384
DIM_HEAD = 64
DIM_MID = 32
DIM_TAIL = 16

_INFO = plsc.get_sparse_core_info()
NC, NS = _INFO.num_cores, _INFO.num_subcores
NW = NC * NS
N_PER = BATCH // NW
CH = 32
NSTEP = N_PER // 16


def _body(x_hbm, g_hbm, head_hbm, mt_hbm, out_hbm,
          xv, gv, xk0, pk0, xk1, pk1, xk2, pk2,
          gb_head, gb_mid, gb_tail, obuf, sem0, sem1, sem2):
    wid = lax.axis_index("s") * NC + lax.axis_index("c")
    base = wid * N_PER
    pltpu.sync_copy(x_hbm.at[pl.ds(base, N_PER)], xv)
    pltpu.sync_copy(g_hbm.at[pl.ds(base, N_PER)], gv)

    zi = jnp.zeros((16,), jnp.int32)
    zf = jnp.zeros((16,), jnp.float32)

    for i in range(NSTEP):
        xk0[pl.ds(i * 16, 16)] = zi
        xk1[pl.ds(i * 16, 16)] = zi
        xk2[pl.ds(i * 16, 16)] = zi

    iota = lax.iota(jnp.int32, 16)
    offs = [jnp.int32(0), jnp.int32(0), jnp.int32(0)]
    for c in range(NSTEP):
        xc = xv[pl.ds(c * 16, 16)]
        gc = gv[pl.ds(c * 16, 16)]
        posc = iota + (c * 16)
        for k, (xk, pk) in enumerate(((xk0, pk0), (xk1, pk1), (xk2, pk2))):
            m = gc == k
            ones = m.astype(jnp.int32)
            incl = plsc.cumsum(ones)
            dest = offs[k] + incl - ones
            plsc.store_scatter(xk, [dest], xc, mask=m)
            plsc.store_scatter(pk, [dest], posc, mask=m)
            offs[k] = offs[k] + jnp.sum(ones)

    def fire_all(nk, xk, tbl, gbuf, sem):
        trips = lax.shift_right_logical(nk + (CH - 1), 5)

        def fire(j, carry):
            pltpu.async_copy(tbl.at[xk.at[pl.ds(j * CH, CH)]],
                             gbuf.at[pl.ds(j * CH, CH)], sem)
            return carry

        lax.fori_loop(0, trips, fire, jnp.int32(0))
        return trips

    def drain_all(trips, xk, tbl, gbuf, sem):
        def drain(j, carry):
            pltpu.make_async_copy(tbl.at[xk.at[pl.ds(j * CH, CH)]],
                                  gbuf.at[pl.ds(j * CH, CH)], sem).wait()
            return carry

        lax.fori_loop(0, trips, drain, jnp.int32(0))

    t0 = fire_all(offs[0], xk0, head_hbm, gb_head, sem0)
    t1 = fire_all(offs[1], xk1, mt_hbm, gb_mid, sem1)
    t2 = fire_all(offs[2], xk2, mt_hbm, gb_tail, sem2)

    def reloc(nk, pk, emit16):
        nsteps = lax.shift_right_logical(nk + 15, 4)

        def step(j, carry):
            rows = iota + j * 16
            mask = rows < nk
            pos = plsc.load_gather(pk, [rows])
            pos = lax.bitwise_and(pos, N_PER - 1)
            emit16(rows, pos, mask)
            return carry

        lax.fori_loop(0, nsteps, step, jnp.int32(0))

    def head16(rows, pos, mask):
        for c in range(DIM_HEAD):
            cv = jnp.full((16,), c, jnp.int32)
            v = plsc.load_gather(gb_head, [rows, cv])
            plsc.store_scatter(obuf, [pos, cv], v, mask=mask)

    def mid16(rows, pos, mask):
        for c in range(DIM_MID):
            cv = jnp.full((16,), c, jnp.int32)
            v = plsc.load_gather(gb_mid, [rows, cv])
            plsc.store_scatter(obuf, [pos, cv], v, mask=mask)
        for c in range(DIM_MID, DIM_HEAD):
            cv = jnp.full((16,), c, jnp.int32)
            plsc.store_scatter(obuf, [pos, cv], zf, mask=mask)

    def tail16(rows, pos, mask):
        for c in range(DIM_TAIL):
            cv = jnp.full((16,), c, jnp.int32)
            ct = jnp.full((16,), DIM_MID + c, jnp.int32)
            v = plsc.load_gather(gb_tail, [rows, ct])
            for q in range(DIM_HEAD // DIM_TAIL):
                cq = jnp.full((16,), c + q * DIM_TAIL, jnp.int32)
                plsc.store_scatter(obuf, [pos, cq], v, mask=mask)

    drain_all(t0, xk0, head_hbm, gb_head, sem0)
    reloc(offs[0], pk0, head16)
    drain_all(t1, xk1, mt_hbm, gb_mid, sem1)
    reloc(offs[1], pk1, mid16)
    drain_all(t2, xk2, mt_hbm, gb_tail, sem2)
    reloc(offs[2], pk2, tail16)

    pltpu.sync_copy(obuf, out_hbm.at[pl.ds(base, N_PER)])


@jax.jit
def _sc_lookup(x, g, head_table, mt_table):
    mesh = plsc.VectorSubcoreMesh(core_axis_name="c", subcore_axis_name="s")
    f = functools.partial(
        pl.kernel,
        mesh=mesh,
        compiler_params=pltpu.CompilerParams(
            needs_layout_passes=False, use_tc_tiling_on_sc=False),
        out_type=jax.ShapeDtypeStruct((BATCH, DIM_HEAD), jnp.float32),
        scratch_types=[
            pltpu.VMEM((N_PER,), jnp.int32),
            pltpu.VMEM((N_PER,), jnp.int32),
            pltpu.VMEM((N_PER,), jnp.int32),
            pltpu.VMEM((N_PER,), jnp.int32),
            pltpu.VMEM((N_PER,), jnp.int32),
            pltpu.VMEM((N_PER,), jnp.int32),
            pltpu.VMEM((N_PER,), jnp.int32),
            pltpu.VMEM((N_PER,), jnp.int32),
            pltpu.VMEM((N_PER, DIM_HEAD), jnp.float32),
            pltpu.VMEM((N_PER, DIM_MID + DIM_TAIL), jnp.float32),
            pltpu.VMEM((N_PER, DIM_MID + DIM_TAIL), jnp.float32),
            pltpu.VMEM((N_PER, DIM_HEAD), jnp.float32),
            pltpu.SemaphoreType.DMA,
            pltpu.SemaphoreType.DMA,
            pltpu.SemaphoreType.DMA,
        ],
    )(_body)
    return f(x, g, head_table, mt_table)


def kernel(x, frequency_groups, head_table, mid_table, tail_table):
    mt = jnp.concatenate([mid_table[:100000], tail_table], axis=1)
    return _sc_lookup(x.astype(jnp.int32), frequency_groups.astype(jnp.int32),
                      head_table, mt)


# speedup vs baseline: 1.5424x; 1.0502x over previous
import functools

import jax
import jax.numpy as jnp
from jax import lax
from jax.experimental import pallas as pl
from jax.experimental.pallas import tpu as pltpu
from jax.experimental.pallas import tpu_sc as plsc

BATCH = 16384
DIM_HEAD = 64
DIM_MID = 32
DIM_TAIL = 16

_INFO = plsc.get_sparse_core_info()
NC, NS = _INFO.num_cores, _INFO.num_subcores
NW = NC * NS
N_PER = BATCH // NW
CH = 32
NSTEP = N_PER // 16


def _body(x_hbm, g_hbm, head_hbm, mid_hbm, tail_hbm, out_hbm,
          xv, gv, xk0, pk0, xk1, pk1, xk2, pk2,
          gb_head, gb_mid, gb_tail, obuf, sem0, sem1, sem2):
    wid = lax.axis_index("s") * NC + lax.axis_index("c")
    base = wid * N_PER
    pltpu.sync_copy(x_hbm.at[pl.ds(base, N_PER)], xv)
    pltpu.sync_copy(g_hbm.at[pl.ds(base, N_PER)], gv)

    zi = jnp.zeros((16,), jnp.int32)
    zf = jnp.zeros((16,), jnp.float32)

    for i in range(NSTEP):
        xk0[pl.ds(i * 16, 16)] = zi
        xk1[pl.ds(i * 16, 16)] = zi
        xk2[pl.ds(i * 16, 16)] = zi

    iota = lax.iota(jnp.int32, 16)
    offs = [jnp.int32(0), jnp.int32(0), jnp.int32(0)]
    for c in range(NSTEP):
        xc = xv[pl.ds(c * 16, 16)]
        gc = gv[pl.ds(c * 16, 16)]
        posc = iota + (c * 16)
        for k, (xk, pk) in enumerate(((xk0, pk0), (xk1, pk1), (xk2, pk2))):
            m = gc == k
            ones = m.astype(jnp.int32)
            incl = plsc.cumsum(ones)
            dest = offs[k] + incl - ones
            plsc.store_scatter(xk, [dest], xc, mask=m)
            plsc.store_scatter(pk, [dest], posc, mask=m)
            offs[k] = offs[k] + jnp.sum(ones)

    def fire_all(nk, xk, tbl, gbuf, sem):
        trips = lax.shift_right_logical(nk + (CH - 1), 5)

        def fire(j, carry):
            pltpu.async_copy(tbl.at[xk.at[pl.ds(j * CH, CH)]],
                             gbuf.at[pl.ds(j * CH, CH)], sem)
            return carry

        lax.fori_loop(0, trips, fire, jnp.int32(0))
        return trips

    def drain_all(trips, xk, tbl, gbuf, sem):
        def drain(j, carry):
            pltpu.make_async_copy(tbl.at[xk.at[pl.ds(j * CH, CH)]],
                                  gbuf.at[pl.ds(j * CH, CH)], sem).wait()
            return carry

        lax.fori_loop(0, trips, drain, jnp.int32(0))

    t0 = fire_all(offs[0], xk0, head_hbm, gb_head, sem0)
    t1 = fire_all(offs[1], xk1, mid_hbm, gb_mid, sem1)
    t2 = fire_all(offs[2], xk2, tail_hbm, gb_tail, sem2)

    def reloc(nk, pk, emit16):
        nsteps = lax.shift_right_logical(nk + 15, 4)

        def step(j, carry):
            rows = iota + j * 16
            mask = rows < nk
            pos = plsc.load_gather(pk, [rows])
            pos = lax.bitwise_and(pos, N_PER - 1)
            emit16(rows, pos, mask)
            return carry

        lax.fori_loop(0, nsteps, step, jnp.int32(0))

    def head16(rows, pos, mask):
        for c in range(DIM_HEAD):
            cv = jnp.full((16,), c, jnp.int32)
            v = plsc.load_gather(gb_head, [rows, cv])
            plsc.store_scatter(obuf, [pos, cv], v, mask=mask)

    def mid16(rows, pos, mask):
        for c in range(DIM_MID):
            cv = jnp.full((16,), c, jnp.int32)
            v = plsc.load_gather(gb_mid, [rows, cv])
            plsc.store_scatter(obuf, [pos, cv], v, mask=mask)
        for c in range(DIM_MID, DIM_HEAD):
            cv = jnp.full((16,), c, jnp.int32)
            plsc.store_scatter(obuf, [pos, cv], zf, mask=mask)

    def tail16(rows, pos, mask):
        for c in range(DIM_TAIL):
            cv = jnp.full((16,), c, jnp.int32)
            v = plsc.load_gather(gb_tail, [rows, cv])
            for q in range(DIM_HEAD // DIM_TAIL):
                cq = jnp.full((16,), c + q * DIM_TAIL, jnp.int32)
                plsc.store_scatter(obuf, [pos, cq], v, mask=mask)

    drain_all(t0, xk0, head_hbm, gb_head, sem0)
    reloc(offs[0], pk0, head16)
    drain_all(t1, xk1, mid_hbm, gb_mid, sem1)
    reloc(offs[1], pk1, mid16)
    drain_all(t2, xk2, tail_hbm, gb_tail, sem2)
    reloc(offs[2], pk2, tail16)

    pltpu.sync_copy(obuf, out_hbm.at[pl.ds(base, N_PER)])


@jax.jit
def _sc_lookup(x, g, head_table, mid_table, tail_table):
    mesh = plsc.VectorSubcoreMesh(core_axis_name="c", subcore_axis_name="s")
    f = functools.partial(
        pl.kernel,
        mesh=mesh,
        compiler_params=pltpu.CompilerParams(
            needs_layout_passes=False, use_tc_tiling_on_sc=False),
        out_type=jax.ShapeDtypeStruct((BATCH, DIM_HEAD), jnp.float32),
        scratch_types=[
            pltpu.VMEM((N_PER,), jnp.int32),
            pltpu.VMEM((N_PER,), jnp.int32),
            pltpu.VMEM((N_PER,), jnp.int32),
            pltpu.VMEM((N_PER,), jnp.int32),
            pltpu.VMEM((N_PER,), jnp.int32),
            pltpu.VMEM((N_PER,), jnp.int32),
            pltpu.VMEM((N_PER,), jnp.int32),
            pltpu.VMEM((N_PER,), jnp.int32),
            pltpu.VMEM((N_PER, DIM_HEAD), jnp.float32),
            pltpu.VMEM((N_PER, DIM_MID), jnp.float32),
            pltpu.VMEM((N_PER, DIM_TAIL), jnp.float32),
            pltpu.VMEM((N_PER, DIM_HEAD), jnp.float32),
            pltpu.SemaphoreType.DMA,
            pltpu.SemaphoreType.DMA,
            pltpu.SemaphoreType.DMA,
        ],
    )(_body)
    return f(x, g, head_table, mid_table, tail_table)


def kernel(x, frequency_groups, head_table, mid_table, tail_table):
    return _sc_lookup(x.astype(jnp.int32), frequency_groups.astype(jnp.int32),
                      head_table, mid_table[:100000], tail_table)
